# baseline (device time: 197606 ns/iter reference)
import functools

import jax
import jax.numpy as jnp
from jax import lax
from jax.experimental import pallas as pl
from jax.experimental.pallas import tpu as pltpu

N_DEV = 4
SQ = 1024
SKV = 1024
HQ_LOCAL = 8
DH = 128
BLK = 64
SCALE = 0.08838834764831843


def _ring_allreduce(partial):
    m, n = partial.shape

    def body(x_ref, out_ref, comm_ref, send_sems, recv_sems):
        my = lax.axis_index("i")
        left = lax.rem(my + (N_DEV - 1), N_DEV)
        right = lax.rem(my + 1, N_DEV)

        barrier_sem = pltpu.get_barrier_semaphore()
        for nbr in (left, right):
            pl.semaphore_signal(
                barrier_sem, inc=1,
                device_id=(nbr,), device_id_type=pl.DeviceIdType.MESH,
            )
        pl.semaphore_wait(barrier_sem, 2)

        comm_ref[0] = x_ref[...]
        out_ref[...] = x_ref[...]

        for h in range(N_DEV - 1):
            rdma = pltpu.make_async_remote_copy(
                src_ref=comm_ref.at[h],
                dst_ref=comm_ref.at[h + 1],
                send_sem=send_sems.at[h],
                recv_sem=recv_sems.at[h],
                device_id=(right,),
                device_id_type=pl.DeviceIdType.MESH,
            )
            rdma.start()
            rdma.wait()
            out_ref[...] += comm_ref[h + 1]

    return pl.pallas_call(
        body,
        out_shape=jax.ShapeDtypeStruct((m, n), jnp.float32),
        in_specs=[pl.BlockSpec(memory_space=pltpu.VMEM)],
        out_specs=pl.BlockSpec(memory_space=pltpu.VMEM),
        scratch_shapes=[
            pltpu.VMEM((N_DEV, m, n), jnp.float32),
            pltpu.SemaphoreType.DMA((N_DEV - 1,)),
            pltpu.SemaphoreType.DMA((N_DEV - 1,)),
        ],
        compiler_params=pltpu.CompilerParams(collective_id=0),
    )(partial)


def kernel(x, Wq, K_ext, V_ext, Wo):
    my = lax.axis_index("i")
    bf16 = jnp.bfloat16

    Wq_l = lax.dynamic_slice(Wq, (0, my * (HQ_LOCAL * DH)), (SQ, HQ_LOCAL * DH))
    xq = x[0].astype(bf16) @ Wq_l.astype(bf16)
    Q = xq.astype(bf16).reshape(SQ, HQ_LOCAL, DH)
    K = K_ext[0].astype(bf16)
    V = V_ext[0].astype(bf16)

    qb = jnp.arange(SQ) // BLK
    kb = jnp.arange(SKV) // BLK
    mask = (
        (qb[:, None] == kb[None, :])
        | (kb[None, :] == 0)
        | ((qb[:, None] + kb[None, :]) % 3 == 0)
    )

    scores = jnp.einsum(
        "ihd,jhd->hij", Q, K, preferred_element_type=jnp.float32
    ) * SCALE
    scores = jnp.where(mask[None, :, :], scores, -1e9)
    w = jax.nn.softmax(scores, axis=-1)
    ctx = jnp.einsum(
        "hij,jhd->ihd", w.astype(bf16), V, preferred_element_type=jnp.float32
    ).reshape(SQ, HQ_LOCAL * DH)

    Wo_l = lax.dynamic_slice(Wo, (my * (HQ_LOCAL * DH), 0), (HQ_LOCAL * DH, SQ))
    partial = jnp.dot(
        ctx.astype(bf16), Wo_l.astype(bf16), preferred_element_type=jnp.float32
    )

    out = _ring_allreduce(partial)
    return out[None, :, :]


# device time: 73319 ns/iter; 2.6952x vs baseline; 2.6952x over previous
import jax
import jax.numpy as jnp
from jax import lax
from jax.experimental import pallas as pl
from jax.experimental.pallas import tpu as pltpu

N_DEV = 4
SQ = 1024
SKV = 1024
HQ_LOCAL = 8
DH = 128
BLK = 64
SCALE = 0.08838834764831843
CH = SQ // N_DEV

_BF16 = jnp.bfloat16
_F32 = jnp.float32


def _compute_chunk(c, x_ref, wq_ref, k_ref, v_ref, wo_ref):
    xc = x_ref[pl.ds(c * CH, CH), :]
    qc = jnp.dot(xc, wq_ref[...], preferred_element_type=_F32).astype(_BF16)

    rows = c * CH + lax.broadcasted_iota(jnp.int32, (CH, SKV), 0)
    cols = lax.broadcasted_iota(jnp.int32, (CH, SKV), 1)
    qb = rows // BLK
    kb = cols // BLK
    mask = (qb == kb) | (kb == 0) | ((qb + kb) % 3 == 0)
    bias = jnp.where(mask, 0.0, -1e9).astype(_F32)

    ctx_parts = []
    for h in range(HQ_LOCAL):
        qh = qc[:, h * DH:(h + 1) * DH]
        kh = k_ref[:, h * DH:(h + 1) * DH]
        sc = lax.dot_general(
            qh, kh, (((1,), (1,)), ((), ())), preferred_element_type=_F32
        ) * SCALE + bias
        m = jnp.max(sc, axis=-1, keepdims=True)
        e = jnp.exp(sc - m)
        w = (e / jnp.sum(e, axis=-1, keepdims=True)).astype(_BF16)
        vh = v_ref[:, h * DH:(h + 1) * DH]
        ctx_parts.append(jnp.dot(w, vh, preferred_element_type=_F32))
    ctx = jnp.concatenate(ctx_parts, axis=1).astype(_BF16)
    return jnp.dot(ctx, wo_ref[...], preferred_element_type=_F32)


def _body(x_ref, wq_ref, k_ref, v_ref, wo_ref, out_ref,
          sbuf, rbuf, agb, rs_send, rs_recv, ag_send, ag_recv):
    my = lax.axis_index("i")
    left = lax.rem(my + (N_DEV - 1), N_DEV)
    right = lax.rem(my + 1, N_DEV)

    barrier_sem = pltpu.get_barrier_semaphore()
    for nbr in (left, right):
        pl.semaphore_signal(
            barrier_sem, inc=1,
            device_id=(nbr,), device_id_type=pl.DeviceIdType.MESH,
        )
    pl.semaphore_wait(barrier_sem, 2)

    acc0 = _compute_chunk(my, x_ref, wq_ref, k_ref, v_ref, wo_ref)
    sbuf[0] = acc0.astype(_BF16)
    for s in range(N_DEV - 1):
        rdma = pltpu.make_async_remote_copy(
            src_ref=sbuf.at[s],
            dst_ref=rbuf.at[s],
            send_sem=rs_send.at[s],
            recv_sem=rs_recv.at[s],
            device_id=(right,),
            device_id_type=pl.DeviceIdType.MESH,
        )
        rdma.start()
        c_next = lax.rem(my + (N_DEV - 1 - s), N_DEV)
        local = _compute_chunk(c_next, x_ref, wq_ref, k_ref, v_ref, wo_ref)
        rdma.wait()
        total = local + rbuf[s].astype(_F32)
        if s < N_DEV - 2:
            sbuf[s + 1] = total.astype(_BF16)
        else:
            out_ref[pl.ds(c_next * CH, CH), :] = total
            agb[0] = total.astype(_BF16)

    for t in range(N_DEV - 1):
        rdma = pltpu.make_async_remote_copy(
            src_ref=agb.at[t],
            dst_ref=agb.at[t + 1],
            send_sem=ag_send.at[t],
            recv_sem=ag_recv.at[t],
            device_id=(right,),
            device_id_type=pl.DeviceIdType.MESH,
        )
        rdma.start()
        rdma.wait()
        ridx = lax.rem(my + (N_DEV - t), N_DEV)
        out_ref[pl.ds(ridx * CH, CH), :] = agb[t + 1].astype(_F32)


def kernel(x, Wq, K_ext, V_ext, Wo):
    my = lax.axis_index("i")
    Wq_l = lax.dynamic_slice(
        Wq, (0, my * (HQ_LOCAL * DH)), (SQ, HQ_LOCAL * DH)
    ).astype(_BF16)
    Wo_l = lax.dynamic_slice(
        Wo, (my * (HQ_LOCAL * DH), 0), (HQ_LOCAL * DH, SQ)
    ).astype(_BF16)
    xb = x[0].astype(_BF16)
    Kb = K_ext[0].astype(_BF16).reshape(SKV, HQ_LOCAL * DH)
    Vb = V_ext[0].astype(_BF16).reshape(SKV, HQ_LOCAL * DH)

    out = pl.pallas_call(
        _body,
        out_shape=jax.ShapeDtypeStruct((SQ, SQ), _F32),
        in_specs=[pl.BlockSpec(memory_space=pltpu.VMEM)] * 5,
        out_specs=pl.BlockSpec(memory_space=pltpu.VMEM),
        scratch_shapes=[
            pltpu.VMEM((N_DEV - 1, CH, SQ), _BF16),
            pltpu.VMEM((N_DEV - 1, CH, SQ), _BF16),
            pltpu.VMEM((N_DEV, CH, SQ), _BF16),
            pltpu.SemaphoreType.DMA((N_DEV - 1,)),
            pltpu.SemaphoreType.DMA((N_DEV - 1,)),
            pltpu.SemaphoreType.DMA((N_DEV - 1,)),
            pltpu.SemaphoreType.DMA((N_DEV - 1,)),
        ],
        compiler_params=pltpu.CompilerParams(collective_id=0),
    )(xb, Wq_l, Kb, Vb, Wo_l)
    return out[None, :, :]


# device time: 57042 ns/iter; 3.4642x vs baseline; 1.2854x over previous
import numpy as np

import jax
import jax.numpy as jnp
from jax import lax
from jax.experimental import pallas as pl
from jax.experimental.pallas import tpu as pltpu

N_DEV = 4
SQ = 1024
SKV = 1024
HQ_LOCAL = 8
DH = 128
BLK = 64
SCALE = 0.08838834764831843
CH = SQ // N_DEV
HN = SQ // 2

_BF16 = jnp.bfloat16
_F32 = jnp.float32

_qb = np.arange(SQ) // BLK
_kb = np.arange(SKV) // BLK
_MASK = (
    (_qb[:, None] == _kb[None, :])
    | (_kb[None, :] == 0)
    | ((_qb[:, None] + _kb[None, :]) % 3 == 0)
)
_BIAS = np.where(_MASK, 0.0, -1e9).astype(np.float32)


def _compute_chunk(c, x_ref, wq_ref, k_ref, v_ref, wo_ref, bias_ref):
    xc = x_ref[pl.ds(c * CH, CH), :]
    qc = jnp.dot(xc, wq_ref[...], preferred_element_type=_F32).astype(_BF16)
    biasc = bias_ref[pl.ds(c * CH, CH), :]

    ctx_parts = []
    for h in range(HQ_LOCAL):
        qh = qc[:, h * DH:(h + 1) * DH]
        kh = k_ref[:, h * DH:(h + 1) * DH]
        sc = lax.dot_general(
            qh, kh, (((1,), (1,)), ((), ())), preferred_element_type=_F32
        )
        e = jnp.exp(sc + biasc)
        w = (e / jnp.sum(e, axis=-1, keepdims=True)).astype(_BF16)
        vh = v_ref[:, h * DH:(h + 1) * DH]
        ctx_parts.append(jnp.dot(w, vh, preferred_element_type=_F32))
    ctx = jnp.concatenate(ctx_parts, axis=1).astype(_BF16)
    return jnp.dot(ctx, wo_ref[...], preferred_element_type=_F32)


def _body(x_ref, wq_ref, k_ref, v_ref, wo_ref, bias_ref, out_ref,
          scw, rcw, sccw, rccw, agcw, agccw,
          rs_send, rs_recv, ag_send, ag_recv):
    my = lax.axis_index("i")
    left = lax.rem(my + (N_DEV - 1), N_DEV)
    right = lax.rem(my + 1, N_DEV)

    barrier_sem = pltpu.get_barrier_semaphore()
    for nbr in (left, right):
        pl.semaphore_signal(
            barrier_sem, inc=1,
            device_id=(nbr,), device_id_type=pl.DeviceIdType.MESH,
        )
    pl.semaphore_wait(barrier_sem, 2)

    def rs_pair(s):
        cw = pltpu.make_async_remote_copy(
            src_ref=scw.at[s], dst_ref=rcw.at[s],
            send_sem=rs_send.at[0, s], recv_sem=rs_recv.at[0, s],
            device_id=(right,), device_id_type=pl.DeviceIdType.MESH,
        )
        ccw = pltpu.make_async_remote_copy(
            src_ref=sccw.at[s], dst_ref=rccw.at[s],
            send_sem=rs_send.at[1, s], recv_sem=rs_recv.at[1, s],
            device_id=(left,), device_id_type=pl.DeviceIdType.MESH,
        )
        return cw, ccw

    args = (x_ref, wq_ref, k_ref, v_ref, wo_ref, bias_ref)

    c0 = _compute_chunk(my, *args)
    scw[0] = c0[:, :HN].astype(_BF16)
    sccw[0] = c0[:, HN:].astype(_BF16)
    cw0, ccw0 = rs_pair(0)
    cw0.start()
    ccw0.start()

    c_cwl = _compute_chunk(lax.rem(my + 3, N_DEV), *args)
    c_ccwl = _compute_chunk(lax.rem(my + 1, N_DEV), *args)
    cw0.wait()
    ccw0.wait()
    scw[1] = (c_cwl[:, :HN] + rcw[0].astype(_F32)).astype(_BF16)
    sccw[1] = (c_ccwl[:, HN:] + rccw[0].astype(_F32)).astype(_BF16)
    cw1, ccw1 = rs_pair(1)
    cw1.start()
    ccw1.start()

    c2 = _compute_chunk(lax.rem(my + 2, N_DEV), *args)
    cw1.wait()
    ccw1.wait()
    scw[2] = (c2[:, :HN] + rcw[1].astype(_F32)).astype(_BF16)
    sccw[2] = (c2[:, HN:] + rccw[1].astype(_F32)).astype(_BF16)
    cw2, ccw2 = rs_pair(2)
    cw2.start()
    ccw2.start()
    cw2.wait()
    ccw2.wait()

    owned_a = rcw[2].astype(_F32) + c_ccwl[:, :HN]
    owned_b = rccw[2].astype(_F32) + c_cwl[:, HN:]
    out_ref[pl.ds(lax.rem(my + 1, N_DEV) * CH, CH), 0:HN] = owned_a
    out_ref[pl.ds(lax.rem(my + 3, N_DEV) * CH, CH), HN:SQ] = owned_b
    agcw[0] = owned_a.astype(_BF16)
    agccw[0] = owned_b.astype(_BF16)

    for t in range(N_DEV - 1):
        cw = pltpu.make_async_remote_copy(
            src_ref=agcw.at[t], dst_ref=agcw.at[t + 1],
            send_sem=ag_send.at[0, t], recv_sem=ag_recv.at[0, t],
            device_id=(right,), device_id_type=pl.DeviceIdType.MESH,
        )
        ccw = pltpu.make_async_remote_copy(
            src_ref=agccw.at[t], dst_ref=agccw.at[t + 1],
            send_sem=ag_send.at[1, t], recv_sem=ag_recv.at[1, t],
            device_id=(left,), device_id_type=pl.DeviceIdType.MESH,
        )
        cw.start()
        ccw.start()
        cw.wait()
        ccw.wait()
        ra = lax.rem(my + (N_DEV - t), N_DEV)
        rb = lax.rem(my + t, N_DEV)
        out_ref[pl.ds(ra * CH, CH), 0:HN] = agcw[t + 1].astype(_F32)
        out_ref[pl.ds(rb * CH, CH), HN:SQ] = agccw[t + 1].astype(_F32)


def kernel(x, Wq, K_ext, V_ext, Wo):
    my = lax.axis_index("i")
    Wq_l = (
        lax.dynamic_slice(Wq, (0, my * (HQ_LOCAL * DH)), (SQ, HQ_LOCAL * DH))
        * SCALE
    ).astype(_BF16)
    Wo_l = lax.dynamic_slice(
        Wo, (my * (HQ_LOCAL * DH), 0), (HQ_LOCAL * DH, SQ)
    ).astype(_BF16)
    xb = x[0].astype(_BF16)
    Kb = K_ext[0].astype(_BF16).reshape(SKV, HQ_LOCAL * DH)
    Vb = V_ext[0].astype(_BF16).reshape(SKV, HQ_LOCAL * DH)
    bias = jnp.asarray(_BIAS)

    out = pl.pallas_call(
        _body,
        out_shape=jax.ShapeDtypeStruct((SQ, SQ), _F32),
        in_specs=[pl.BlockSpec(memory_space=pltpu.VMEM)] * 6,
        out_specs=pl.BlockSpec(memory_space=pltpu.VMEM),
        scratch_shapes=[
            pltpu.VMEM((N_DEV - 1, CH, HN), _BF16),
            pltpu.VMEM((N_DEV - 1, CH, HN), _BF16),
            pltpu.VMEM((N_DEV - 1, CH, HN), _BF16),
            pltpu.VMEM((N_DEV - 1, CH, HN), _BF16),
            pltpu.VMEM((N_DEV, CH, HN), _BF16),
            pltpu.VMEM((N_DEV, CH, HN), _BF16),
            pltpu.SemaphoreType.DMA((2, N_DEV - 1)),
            pltpu.SemaphoreType.DMA((2, N_DEV - 1)),
            pltpu.SemaphoreType.DMA((2, N_DEV - 1)),
            pltpu.SemaphoreType.DMA((2, N_DEV - 1)),
        ],
        compiler_params=pltpu.CompilerParams(collective_id=0),
    )(xb, Wq_l, Kb, Vb, Wo_l, bias)
    return out[None, :, :]


# device time: 52538 ns/iter; 3.7612x vs baseline; 1.0857x over previous
import numpy as np

import jax
import jax.numpy as jnp
from jax import lax
from jax.experimental import pallas as pl
from jax.experimental.pallas import tpu as pltpu

N_DEV = 4
SQ = 1024
SKV = 1024
HQ_LOCAL = 8
DH = 128
BLK = 64
SCALE = 0.08838834764831843
LOG2E = 1.4426950408889634
CH = SQ // N_DEV
HN = SQ // 2

_BF16 = jnp.bfloat16
_F32 = jnp.float32

_qb = np.arange(SQ) // BLK
_kb = np.arange(SKV) // BLK
_MASK = (
    (_qb[:, None] == _kb[None, :])
    | (_kb[None, :] == 0)
    | ((_qb[:, None] + _kb[None, :]) % 3 == 0)
)
_BIAS = np.where(_MASK, 0.0, -1e9).astype(np.float32)


def _compute_chunk(c, x_ref, wq_ref, k_ref, v_ref, wo_ref, bias_ref):
    xc = x_ref[pl.ds(c * CH, CH), :]
    qc = jnp.dot(xc, wq_ref[...], preferred_element_type=_F32).astype(_BF16)
    biasc = bias_ref[pl.ds(c * CH, CH), :]

    ctx_parts = []
    for h in range(HQ_LOCAL):
        qh = qc[:, h * DH:(h + 1) * DH]
        kh = k_ref[:, h * DH:(h + 1) * DH]
        sc = lax.dot_general(
            qh, kh, (((1,), (1,)), ((), ())), preferred_element_type=_F32
        )
        e = jnp.exp2(sc + biasc)
        recip = 1.0 / jnp.sum(e, axis=-1, keepdims=True)
        vh = v_ref[:, h * DH:(h + 1) * DH]
        av = jnp.dot(e.astype(_BF16), vh, preferred_element_type=_F32)
        ctx_parts.append(av * recip)
    ctx = jnp.concatenate(ctx_parts, axis=1).astype(_BF16)
    return jnp.dot(ctx, wo_ref[...], preferred_element_type=_F32)


def _body(x_ref, wq_ref, k_ref, v_ref, wo_ref, bias_ref, out_ref,
          scw, rcw, sccw, rccw, agcw, agccw,
          rs_send, rs_recv, ag_send, ag_recv):
    my = lax.axis_index("i")
    left = lax.rem(my + (N_DEV - 1), N_DEV)
    right = lax.rem(my + 1, N_DEV)

    barrier_sem = pltpu.get_barrier_semaphore()
    for nbr in (left, right):
        pl.semaphore_signal(
            barrier_sem, inc=1,
            device_id=(nbr,), device_id_type=pl.DeviceIdType.MESH,
        )
    pl.semaphore_wait(barrier_sem, 2)

    def rs_pair(s):
        cw = pltpu.make_async_remote_copy(
            src_ref=scw.at[s], dst_ref=rcw.at[s],
            send_sem=rs_send.at[0, s], recv_sem=rs_recv.at[0, s],
            device_id=(right,), device_id_type=pl.DeviceIdType.MESH,
        )
        ccw = pltpu.make_async_remote_copy(
            src_ref=sccw.at[s], dst_ref=rccw.at[s],
            send_sem=rs_send.at[1, s], recv_sem=rs_recv.at[1, s],
            device_id=(left,), device_id_type=pl.DeviceIdType.MESH,
        )
        return cw, ccw

    args = (x_ref, wq_ref, k_ref, v_ref, wo_ref, bias_ref)

    c0 = _compute_chunk(my, *args)
    scw[0] = c0[:, :HN].astype(_BF16)
    sccw[0] = c0[:, HN:].astype(_BF16)
    cw0, ccw0 = rs_pair(0)
    cw0.start()
    ccw0.start()

    c_cwl = _compute_chunk(lax.rem(my + 3, N_DEV), *args)
    c_ccwl = _compute_chunk(lax.rem(my + 1, N_DEV), *args)
    cw0.wait()
    ccw0.wait()
    scw[1] = (c_cwl[:, :HN] + rcw[0].astype(_F32)).astype(_BF16)
    sccw[1] = (c_ccwl[:, HN:] + rccw[0].astype(_F32)).astype(_BF16)
    cw1, ccw1 = rs_pair(1)
    cw1.start()
    ccw1.start()

    c2 = _compute_chunk(lax.rem(my + 2, N_DEV), *args)
    cw1.wait()
    ccw1.wait()
    scw[2] = (c2[:, :HN] + rcw[1].astype(_F32)).astype(_BF16)
    sccw[2] = (c2[:, HN:] + rccw[1].astype(_F32)).astype(_BF16)
    cw2, ccw2 = rs_pair(2)
    cw2.start()
    ccw2.start()
    cw2.wait()
    ccw2.wait()

    owned_a = rcw[2].astype(_F32) + c_ccwl[:, :HN]
    owned_b = rccw[2].astype(_F32) + c_cwl[:, HN:]
    agcw[0] = owned_a.astype(_BF16)
    agccw[0] = owned_b.astype(_BF16)

    for t in range(N_DEV - 1):
        cw = pltpu.make_async_remote_copy(
            src_ref=agcw.at[t], dst_ref=agcw.at[t + 1],
            send_sem=ag_send.at[0, t], recv_sem=ag_recv.at[0, t],
            device_id=(right,), device_id_type=pl.DeviceIdType.MESH,
        )
        ccw = pltpu.make_async_remote_copy(
            src_ref=agccw.at[t], dst_ref=agccw.at[t + 1],
            send_sem=ag_send.at[1, t], recv_sem=ag_recv.at[1, t],
            device_id=(left,), device_id_type=pl.DeviceIdType.MESH,
        )
        cw.start()
        ccw.start()
        if t == 0:
            out_ref[0, pl.ds(lax.rem(my + 1, N_DEV) * CH, CH), 0:HN] = owned_a
            out_ref[0, pl.ds(lax.rem(my + 3, N_DEV) * CH, CH), HN:SQ] = owned_b
        else:
            ra = lax.rem(my + (N_DEV - (t - 1)), N_DEV)
            rb = lax.rem(my + (t - 1), N_DEV)
            out_ref[0, pl.ds(ra * CH, CH), 0:HN] = agcw[t].astype(_F32)
            out_ref[0, pl.ds(rb * CH, CH), HN:SQ] = agccw[t].astype(_F32)
        cw.wait()
        ccw.wait()
    ra = lax.rem(my + 2, N_DEV)
    out_ref[0, pl.ds(ra * CH, CH), 0:HN] = agcw[N_DEV - 1].astype(_F32)
    out_ref[0, pl.ds(ra * CH, CH), HN:SQ] = agccw[N_DEV - 1].astype(_F32)


def kernel(x, Wq, K_ext, V_ext, Wo):
    my = lax.axis_index("i")
    Wq_l = (
        lax.dynamic_slice(Wq, (0, my * (HQ_LOCAL * DH)), (SQ, HQ_LOCAL * DH))
        * (SCALE * LOG2E)
    ).astype(_BF16)
    Wo_l = lax.dynamic_slice(
        Wo, (my * (HQ_LOCAL * DH), 0), (HQ_LOCAL * DH, SQ)
    ).astype(_BF16)
    xb = x[0].astype(_BF16)
    Kb = K_ext[0].astype(_BF16).reshape(SKV, HQ_LOCAL * DH)
    Vb = V_ext[0].astype(_BF16).reshape(SKV, HQ_LOCAL * DH)
    bias = jnp.asarray(_BIAS)

    out = pl.pallas_call(
        _body,
        out_shape=jax.ShapeDtypeStruct((1, SQ, SQ), _F32),
        in_specs=[pl.BlockSpec(memory_space=pltpu.VMEM)] * 6,
        out_specs=pl.BlockSpec(memory_space=pltpu.VMEM),
        scratch_shapes=[
            pltpu.VMEM((N_DEV - 1, CH, HN), _BF16),
            pltpu.VMEM((N_DEV - 1, CH, HN), _BF16),
            pltpu.VMEM((N_DEV - 1, CH, HN), _BF16),
            pltpu.VMEM((N_DEV - 1, CH, HN), _BF16),
            pltpu.VMEM((N_DEV, CH, HN), _BF16),
            pltpu.VMEM((N_DEV, CH, HN), _BF16),
            pltpu.SemaphoreType.DMA((2, N_DEV - 1)),
            pltpu.SemaphoreType.DMA((2, N_DEV - 1)),
            pltpu.SemaphoreType.DMA((2, N_DEV - 1)),
            pltpu.SemaphoreType.DMA((2, N_DEV - 1)),
        ],
        compiler_params=pltpu.CompilerParams(collective_id=0),
    )(xb, Wq_l, Kb, Vb, Wo_l, bias)
    return out
